# Initial kernel scaffold; baseline (speedup 1.0000x reference)
import jax
import jax.numpy as jnp
from jax import lax
from jax.experimental import pallas as pl
from jax.experimental.pallas import tpu as pltpu
from jax.experimental.pallas import tpu_sc as plsc

N = 10000
E = 320000
NPAD = 10240
NC = 2
NS = 16
NW = NC * NS
EPW = E // NW
K = 80
NCHUNK = EPW // K
RPT = N // NS
RB = 1000


def _mesh():
    return plsc.VectorSubcoreMesh(core_axis_name="c", subcore_axis_name="s")


def _deg_body(dst_hbm, ones_hbm, zeros_hbm, out_hbm, ones_v, idx_d, acc):
    c = lax.axis_index("c")
    s = lax.axis_index("s")
    wid = s * NC + c
    rpt = NPAD // NS
    pltpu.sync_copy(zeros_hbm.at[pl.ds(s * rpt, rpt)], acc.at[pl.ds(s * rpt, rpt)])
    pltpu.sync_copy(ones_hbm, ones_v)
    plsc.subcore_barrier()

    def chunk(i, carry):
        base = wid * EPW + i * K
        pltpu.sync_copy(dst_hbm.at[pl.ds(base, K)], idx_d)
        pltpu.sync_copy(ones_v, acc.at[idx_d], add=True)
        return carry

    lax.fori_loop(0, NCHUNK, chunk, 0)
    plsc.subcore_barrier()
    pltpu.sync_copy(acc.at[pl.ds(s * rpt, rpt)], out_hbm.at[c, pl.ds(s * rpt, rpt)])


def _deg_counts(dst):
    ones = jnp.ones((K,), jnp.float32)
    zeros = jnp.zeros((NPAD,), jnp.float32)
    return pl.kernel(
        _deg_body,
        out_type=jax.ShapeDtypeStruct((NC, NPAD), jnp.float32),
        mesh=_mesh(),
        scratch_types=[
            pltpu.VMEM((K,), jnp.float32),
            pltpu.VMEM((K,), jnp.int32),
            pltpu.VMEM_SHARED((NPAD,), jnp.float32),
        ],
    )(dst, ones, zeros)


def _prop_body(y_hbm, src_hbm, dst_hbm, zeros_hbm, out_hbm, idx_s, idx_d, rows, acc, sem):
    c = lax.axis_index("c")
    s = lax.axis_index("s")
    wid = s * NC + c
    pltpu.sync_copy(zeros_hbm.at[pl.ds(s * RPT, RPT), :], acc.at[pl.ds(s * RPT, RPT), :])
    plsc.subcore_barrier()

    def chunk(i, carry):
        base = wid * EPW + i * K
        pltpu.sync_copy(src_hbm.at[pl.ds(base, K)], idx_s)
        pltpu.sync_copy(dst_hbm.at[pl.ds(base, K)], idx_d)
        pltpu.async_copy(y_hbm.at[idx_s], rows, sem).wait()
        pltpu.sync_copy(rows, acc.at[idx_d], add=True)
        return carry

    lax.fori_loop(0, NCHUNK, chunk, 0)
    plsc.subcore_barrier()
    pltpu.sync_copy(acc.at[pl.ds(s * RPT, RPT), :], out_hbm.at[c, pl.ds(s * RPT, RPT), :])


def _edge_sum(y, src, dst, d):
    zeros = jnp.zeros((N, d), jnp.float32)
    return pl.kernel(
        _prop_body,
        out_type=jax.ShapeDtypeStruct((NC, N, d), jnp.float32),
        mesh=_mesh(),
        scratch_types=[
            pltpu.VMEM((K,), jnp.int32),
            pltpu.VMEM((K,), jnp.int32),
            pltpu.VMEM((K, d), jnp.float32),
            pltpu.VMEM_SHARED((N, d), jnp.float32),
            pltpu.SemaphoreType.DMA,
        ],
    )(y, src, dst, zeros)


def _tcA_body(cnt_ref, x_ref, w1_ref, out_ref):
    dis = lax.rsqrt(cnt_ref[0] + cnt_ref[1] + 1.0)
    y = jnp.dot(x_ref[...], w1_ref[...], preferred_element_type=jnp.float32)
    out_ref[...] = y * dis[:, None]


def _tcA(cnt, x, w1):
    return pl.pallas_call(
        _tcA_body,
        grid=(N // RB,),
        in_specs=[
            pl.BlockSpec((NC, RB), lambda i: (0, i)),
            pl.BlockSpec((RB, 128), lambda i: (i, 0)),
            pl.BlockSpec((128, 128), lambda i: (0, 0)),
        ],
        out_specs=pl.BlockSpec((RB, 128), lambda i: (i, 0)),
        out_shape=jax.ShapeDtypeStruct((N, 128), jnp.float32),
    )(cnt, x, w1)


def _tcB_body(cnt_ref, g_ref, y1_ref, w2_ref, wl_ref, b1_ref, out_ref):
    dis = lax.rsqrt(cnt_ref[0] + cnt_ref[1] + 1.0)
    p = (g_ref[0] + g_ref[1] + y1_ref[...]) * dis[:, None] + b1_ref[...]
    h = jnp.maximum(p, 0.0)
    y2 = jnp.dot(jnp.dot(h, w2_ref[...], preferred_element_type=jnp.float32),
                 wl_ref[...], preferred_element_type=jnp.float32)
    out_ref[...] = y2 * dis[:, None]


def _tcB(cnt, g1, y1, w2, wl, b1):
    return pl.pallas_call(
        _tcB_body,
        grid=(N // RB,),
        in_specs=[
            pl.BlockSpec((NC, RB), lambda i: (0, i)),
            pl.BlockSpec((NC, RB, 128), lambda i: (0, i, 0)),
            pl.BlockSpec((RB, 128), lambda i: (i, 0)),
            pl.BlockSpec((128, 64), lambda i: (0, 0)),
            pl.BlockSpec((64, 64), lambda i: (0, 0)),
            pl.BlockSpec((1, 128), lambda i: (0, 0)),
        ],
        out_specs=pl.BlockSpec((RB, 64), lambda i: (i, 0)),
        out_shape=jax.ShapeDtypeStruct((N, 64), jnp.float32),
    )(cnt, g1, y1, w2, wl, b1)


def _tcC_body(cnt_ref, g_ref, y2_ref, b2_ref, wl_ref, bl_ref, out_ref):
    dis = lax.rsqrt(cnt_ref[0] + cnt_ref[1] + 1.0)
    bias = jnp.dot(b2_ref[...], wl_ref[...], preferred_element_type=jnp.float32) + bl_ref[...]
    out_ref[...] = (g_ref[0] + g_ref[1] + y2_ref[...]) * dis[:, None] + bias


def _tcC(cnt, g2, y2, b2, wl, bl):
    return pl.pallas_call(
        _tcC_body,
        grid=(N // RB,),
        in_specs=[
            pl.BlockSpec((NC, RB), lambda i: (0, i)),
            pl.BlockSpec((NC, RB, 64), lambda i: (0, i, 0)),
            pl.BlockSpec((RB, 64), lambda i: (i, 0)),
            pl.BlockSpec((1, 64), lambda i: (0, 0)),
            pl.BlockSpec((64, 64), lambda i: (0, 0)),
            pl.BlockSpec((1, 64), lambda i: (0, 0)),
        ],
        out_specs=pl.BlockSpec((RB, 64), lambda i: (i, 0)),
        out_shape=jax.ShapeDtypeStruct((N, 64), jnp.float32),
    )(cnt, g2, y2, b2, wl, bl)


def kernel(x, edge_index, W1, b1, W2, b2, Wl, bl):
    src = edge_index[0]
    dst = edge_index[1]
    cnt = _deg_counts(dst)
    y1t = _tcA(cnt, x, W1)
    g1 = _edge_sum(y1t, src, dst, 128)
    y2t = _tcB(cnt, g1, y1t, W2, Wl, b1.reshape(1, 128))
    g2 = _edge_sum(y2t, src, dst, 64)
    return _tcC(cnt, g2, y2t, b2.reshape(1, 64), Wl, bl.reshape(1, 64))


# speedup vs baseline: 9.6693x; 9.6693x over previous
import jax
import jax.numpy as jnp
from jax import lax
from jax.experimental import pallas as pl
from jax.experimental.pallas import tpu as pltpu
from jax.experimental.pallas import tpu_sc as plsc

N = 10000
E = 320000
NPAD = 10240
NC = 2
NS = 16
NW = NC * NS
EPW = E // NW
K = 80
NCHUNK = EPW // K
RPT = NPAD // NS
RB = 1000


def _mesh():
    return plsc.VectorSubcoreMesh(core_axis_name="c", subcore_axis_name="s")


def _deg_body(dst_hbm, ones_hbm, zeros_hbm, out_hbm, ones_v, idx_d, acc):
    c = lax.axis_index("c")
    s = lax.axis_index("s")
    wid = s * NC + c
    rpt = NPAD // NS
    pltpu.sync_copy(zeros_hbm.at[pl.ds(s * rpt, rpt)], acc.at[pl.ds(s * rpt, rpt)])
    pltpu.sync_copy(ones_hbm, ones_v)
    plsc.subcore_barrier()

    def chunk(i, carry):
        base = wid * EPW + i * K
        pltpu.sync_copy(dst_hbm.at[pl.ds(base, K)], idx_d)
        pltpu.sync_copy(ones_v, acc.at[idx_d], add=True)
        return carry

    lax.fori_loop(0, NCHUNK, chunk, 0)
    plsc.subcore_barrier()
    pltpu.sync_copy(acc.at[pl.ds(s * rpt, rpt)], out_hbm.at[c, pl.ds(s * rpt, rpt)])


def _deg_counts(dst):
    ones = jnp.ones((K,), jnp.float32)
    zeros = jnp.zeros((NPAD,), jnp.float32)
    return pl.kernel(
        _deg_body,
        out_type=jax.ShapeDtypeStruct((NC, NPAD), jnp.float32),
        mesh=_mesh(),
        scratch_types=[
            pltpu.VMEM((K,), jnp.float32),
            pltpu.VMEM((K,), jnp.int32),
            pltpu.VMEM_SHARED((NPAD,), jnp.float32),
        ],
    )(dst, ones, zeros)


def _prop_body(y_hbm, src_hbm, dst_hbm, zeros_hbm, out_hbm, idx_s, idx_d, rows, acc, sem):
    c = lax.axis_index("c")
    s = lax.axis_index("s")
    wid = s * NC + c
    pltpu.sync_copy(zeros_hbm.at[pl.ds(s * RPT, RPT), :], acc.at[pl.ds(s * RPT, RPT), :])
    plsc.subcore_barrier()

    def chunk(i, carry):
        base = wid * EPW + i * K
        pltpu.sync_copy(src_hbm.at[pl.ds(base, K)], idx_s)
        pltpu.sync_copy(dst_hbm.at[pl.ds(base, K)], idx_d)
        pltpu.async_copy(y_hbm.at[idx_s], rows, sem).wait()
        pltpu.sync_copy(rows, acc.at[idx_d], add=True)
        return carry

    lax.fori_loop(0, NCHUNK, chunk, 0)
    plsc.subcore_barrier()
    pltpu.sync_copy(acc.at[pl.ds(s * RPT, RPT), :], out_hbm.at[c, pl.ds(s * RPT, RPT), :])


def _edge_sum(y, src, dst, d):
    zeros = jnp.zeros((NPAD, d), jnp.float32)
    return pl.kernel(
        _prop_body,
        out_type=jax.ShapeDtypeStruct((NC, NPAD, d), jnp.float32),
        mesh=_mesh(),
        scratch_types=[
            pltpu.VMEM((K,), jnp.int32),
            pltpu.VMEM((K,), jnp.int32),
            pltpu.VMEM((K, d), jnp.float32),
            pltpu.VMEM_SHARED((NPAD, d), jnp.float32),
            pltpu.SemaphoreType.DMA,
        ],
    )(y, src, dst, zeros)


def _tcA_body(cnt_ref, x_ref, w1_ref, out_ref):
    dis = lax.rsqrt(cnt_ref[0, 0] + cnt_ref[1, 0] + 1.0)
    y = jnp.dot(x_ref[...], w1_ref[...], preferred_element_type=jnp.float32)
    out_ref[...] = y * dis


def _tcA(cnt, x, w1):
    return pl.pallas_call(
        _tcA_body,
        grid=(N // RB,),
        in_specs=[
            pl.BlockSpec((NC, 1, RB, 1), lambda i: (0, i, 0, 0)),
            pl.BlockSpec((RB, 128), lambda i: (i, 0)),
            pl.BlockSpec((128, 128), lambda i: (0, 0)),
        ],
        out_specs=pl.BlockSpec((RB, 128), lambda i: (i, 0)),
        out_shape=jax.ShapeDtypeStruct((N, 128), jnp.float32),
    )(cnt, x, w1)


def _tcB_body(cnt_ref, g_ref, y1_ref, w2_ref, wl_ref, b1_ref, out_ref):
    dis = lax.rsqrt(cnt_ref[0, 0] + cnt_ref[1, 0] + 1.0)
    p = (g_ref[0] + g_ref[1] + y1_ref[...]) * dis + b1_ref[...]
    h = jnp.maximum(p, 0.0)
    y2 = jnp.dot(jnp.dot(h, w2_ref[...], preferred_element_type=jnp.float32),
                 wl_ref[...], preferred_element_type=jnp.float32)
    out_ref[:, :64] = y2 * dis
    out_ref[:, 64:] = jnp.zeros((RB, 64), jnp.float32)


def _tcB(cnt, g1, y1, w2, wl, b1):
    return pl.pallas_call(
        _tcB_body,
        grid=(N // RB,),
        in_specs=[
            pl.BlockSpec((NC, 1, RB, 1), lambda i: (0, i, 0, 0)),
            pl.BlockSpec((NC, RB, 128), lambda i: (0, i, 0)),
            pl.BlockSpec((RB, 128), lambda i: (i, 0)),
            pl.BlockSpec((128, 64), lambda i: (0, 0)),
            pl.BlockSpec((64, 64), lambda i: (0, 0)),
            pl.BlockSpec((1, 128), lambda i: (0, 0)),
        ],
        out_specs=pl.BlockSpec((RB, 128), lambda i: (i, 0)),
        out_shape=jax.ShapeDtypeStruct((N, 128), jnp.float32),
    )(cnt, g1, y1, w2, wl, b1)


def _tcC_body(cnt_ref, g_ref, y2_ref, b2_ref, wl_ref, bl_ref, out_ref):
    dis = lax.rsqrt(cnt_ref[0, 0] + cnt_ref[1, 0] + 1.0)
    bias = jnp.dot(b2_ref[...], wl_ref[...], preferred_element_type=jnp.float32) + bl_ref[...]
    p = g_ref[0, :, :64] + g_ref[1, :, :64] + y2_ref[:, :64]
    out_ref[...] = p * dis + bias


def _tcC(cnt, g2, y2, b2, wl, bl):
    return pl.pallas_call(
        _tcC_body,
        grid=(N // RB,),
        in_specs=[
            pl.BlockSpec((NC, 1, RB, 1), lambda i: (0, i, 0, 0)),
            pl.BlockSpec((NC, RB, 128), lambda i: (0, i, 0)),
            pl.BlockSpec((RB, 128), lambda i: (i, 0)),
            pl.BlockSpec((1, 64), lambda i: (0, 0)),
            pl.BlockSpec((64, 64), lambda i: (0, 0)),
            pl.BlockSpec((1, 64), lambda i: (0, 0)),
        ],
        out_specs=pl.BlockSpec((RB, 64), lambda i: (i, 0)),
        out_shape=jax.ShapeDtypeStruct((N, 64), jnp.float32),
    )(cnt, g2, y2, b2, wl, bl)


def kernel(x, edge_index, W1, b1, W2, b2, Wl, bl):
    src = edge_index[0]
    dst = edge_index[1]
    cnt = _deg_counts(dst)
    cnt = cnt[:, :N].reshape(NC, N // RB, RB, 1)
    y1t = _tcA(cnt, x, W1)
    g1 = _edge_sum(y1t, src, dst, 128)
    y2t = _tcB(cnt, g1, y1t, W2, Wl, b1.reshape(1, 128))
    g2 = _edge_sum(y2t, src, dst, 128)
    return _tcC(cnt, g2, y2t, b2.reshape(1, 64), Wl, bl.reshape(1, 64))


# speedup vs baseline: 25.3726x; 2.6240x over previous
import jax
import jax.numpy as jnp
from jax import lax
from jax.experimental import pallas as pl
from jax.experimental.pallas import tpu as pltpu
from jax.experimental.pallas import tpu_sc as plsc

N = 10000
E = 320000
NPAD = 10240
NC = 2
NS = 16
NW = NC * NS
EPW = E // NW
K = 40
NCHUNK = EPW // K
RPT = NPAD // NS
RB = 1000
NB = 5


def _mesh():
    return plsc.VectorSubcoreMesh(core_axis_name="c", subcore_axis_name="s")


def _deg_body(dst3_hbm, ones_hbm, zeros_hbm, out_hbm, ones_v, idx_d3, acc, sem_s):
    c = lax.axis_index("c")
    s = lax.axis_index("s")
    wid = s * NC + c
    rpt = NPAD // NS
    pltpu.sync_copy(zeros_hbm.at[pl.ds(s * rpt, rpt)], acc.at[pl.ds(s * rpt, rpt)])
    pltpu.sync_copy(ones_hbm, ones_v)
    pltpu.sync_copy(dst3_hbm.at[wid], idx_d3)
    plsc.subcore_barrier()

    def outer(g, carry):
        for b in range(NB):
            i = g * NB + b

            @pl.when(i >= NB)
            def _():
                pltpu.make_async_copy(
                    ones_v, acc.at[idx_d3.at[pl.ds((i - NB) * K, K)]], sem_s.at[b]).wait()

            pltpu.async_copy(ones_v, acc.at[idx_d3.at[pl.ds(i * K, K)]], sem_s.at[b], add=True)
        return carry

    lax.fori_loop(0, NCHUNK // NB, outer, 0)
    for b in range(NB):
        last = (NCHUNK // NB - 1) * NB + b
        pltpu.make_async_copy(
            ones_v, acc.at[idx_d3.at[pl.ds(last * K, K)]], sem_s.at[b]).wait()
    plsc.subcore_barrier()
    pltpu.sync_copy(acc.at[pl.ds(s * rpt, rpt)], out_hbm.at[c, pl.ds(s * rpt, rpt)])


def _deg_counts(dst3):
    ones = jnp.ones((K,), jnp.float32)
    zeros = jnp.zeros((NPAD,), jnp.float32)
    return pl.kernel(
        _deg_body,
        out_type=jax.ShapeDtypeStruct((NC, NPAD), jnp.float32),
        mesh=_mesh(),
        scratch_types=[
            pltpu.VMEM((K,), jnp.float32),
            pltpu.VMEM((EPW,), jnp.int32),
            pltpu.VMEM_SHARED((NPAD,), jnp.float32),
            pltpu.SemaphoreType.DMA((NB,)),
        ],
    )(dst3, ones, zeros)


def _prop_body(y_hbm, src3_hbm, dst3_hbm, zeros_hbm, out_hbm,
               idx_s3, idx_d3, rows, acc, sem_g, sem_s):
    c = lax.axis_index("c")
    s = lax.axis_index("s")
    wid = s * NC + c
    pltpu.sync_copy(zeros_hbm.at[pl.ds(s * RPT, RPT), :], acc.at[pl.ds(s * RPT, RPT), :])
    pltpu.sync_copy(src3_hbm.at[wid], idx_s3)
    pltpu.sync_copy(dst3_hbm.at[wid], idx_d3)
    plsc.subcore_barrier()

    for b in range(NB):
        pltpu.async_copy(y_hbm.at[idx_s3.at[pl.ds(b * K, K)]], rows.at[b], sem_g.at[b])

    def outer(g, carry):
        for b in range(NB):
            i = g * NB + b
            jb = (b + NB - 1) % NB
            pltpu.make_async_copy(
                y_hbm.at[idx_s3.at[pl.ds(i * K, K)]], rows.at[b], sem_g.at[b]).wait()
            pltpu.async_copy(rows.at[b], acc.at[idx_d3.at[pl.ds(i * K, K)]], sem_s.at[b], add=True)
            j = i + NB - 1

            @pl.when((i >= 1) & (j <= NCHUNK - 1))
            def _():
                pltpu.make_async_copy(
                    rows.at[jb], acc.at[idx_d3.at[pl.ds((j - NB) * K, K)]], sem_s.at[jb]).wait()
                pltpu.async_copy(y_hbm.at[idx_s3.at[pl.ds(j * K, K)]], rows.at[jb], sem_g.at[jb])
        return carry

    lax.fori_loop(0, NCHUNK // NB, outer, 0)
    for b in range(NB):
        last = (NCHUNK // NB - 1) * NB + b
        pltpu.make_async_copy(
            rows.at[b], acc.at[idx_d3.at[pl.ds(last * K, K)]], sem_s.at[b]).wait()
    plsc.subcore_barrier()
    pltpu.sync_copy(acc.at[pl.ds(s * RPT, RPT), :], out_hbm.at[c, pl.ds(s * RPT, RPT), :])


def _edge_sum(y, src3, dst3, d):
    zeros = jnp.zeros((NPAD, d), jnp.float32)
    return pl.kernel(
        _prop_body,
        out_type=jax.ShapeDtypeStruct((NC, NPAD, d), jnp.float32),
        mesh=_mesh(),
        scratch_types=[
            pltpu.VMEM((EPW,), jnp.int32),
            pltpu.VMEM((EPW,), jnp.int32),
            pltpu.VMEM((NB, K, d), jnp.float32),
            pltpu.VMEM_SHARED((NPAD, d), jnp.float32),
            pltpu.SemaphoreType.DMA((NB,)),
            pltpu.SemaphoreType.DMA((NB,)),
        ],
    )(y, src3, dst3, zeros)


def _tcA_body(cnt_ref, x_ref, w1_ref, out_ref):
    dis = lax.rsqrt(cnt_ref[0, 0] + cnt_ref[1, 0] + 1.0)
    y = jnp.dot(x_ref[...], w1_ref[...], preferred_element_type=jnp.float32)
    out_ref[...] = y * dis


def _tcA(cnt, x, w1):
    return pl.pallas_call(
        _tcA_body,
        grid=(N // RB,),
        in_specs=[
            pl.BlockSpec((NC, 1, RB, 1), lambda i: (0, i, 0, 0)),
            pl.BlockSpec((RB, 128), lambda i: (i, 0)),
            pl.BlockSpec((128, 128), lambda i: (0, 0)),
        ],
        out_specs=pl.BlockSpec((RB, 128), lambda i: (i, 0)),
        out_shape=jax.ShapeDtypeStruct((N, 128), jnp.float32),
    )(cnt, x, w1)


def _tcB_body(cnt_ref, g_ref, y1_ref, w2_ref, wl_ref, b1_ref, out_ref):
    dis = lax.rsqrt(cnt_ref[0, 0] + cnt_ref[1, 0] + 1.0)
    p = (g_ref[0] + g_ref[1] + y1_ref[...]) * dis + b1_ref[...]
    h = jnp.maximum(p, 0.0)
    y2 = jnp.dot(jnp.dot(h, w2_ref[...], preferred_element_type=jnp.float32),
                 wl_ref[...], preferred_element_type=jnp.float32)
    out_ref[:, :64] = y2 * dis
    out_ref[:, 64:] = jnp.zeros((RB, 64), jnp.float32)


def _tcB(cnt, g1, y1, w2, wl, b1):
    return pl.pallas_call(
        _tcB_body,
        grid=(N // RB,),
        in_specs=[
            pl.BlockSpec((NC, 1, RB, 1), lambda i: (0, i, 0, 0)),
            pl.BlockSpec((NC, RB, 128), lambda i: (0, i, 0)),
            pl.BlockSpec((RB, 128), lambda i: (i, 0)),
            pl.BlockSpec((128, 64), lambda i: (0, 0)),
            pl.BlockSpec((64, 64), lambda i: (0, 0)),
            pl.BlockSpec((1, 128), lambda i: (0, 0)),
        ],
        out_specs=pl.BlockSpec((RB, 128), lambda i: (i, 0)),
        out_shape=jax.ShapeDtypeStruct((N, 128), jnp.float32),
    )(cnt, g1, y1, w2, wl, b1)


def _tcC_body(cnt_ref, g_ref, y2_ref, b2_ref, wl_ref, bl_ref, out_ref):
    dis = lax.rsqrt(cnt_ref[0, 0] + cnt_ref[1, 0] + 1.0)
    bias = jnp.dot(b2_ref[...], wl_ref[...], preferred_element_type=jnp.float32) + bl_ref[...]
    p = g_ref[0, :, :64] + g_ref[1, :, :64] + y2_ref[:, :64]
    out_ref[...] = p * dis + bias


def _tcC(cnt, g2, y2, b2, wl, bl):
    return pl.pallas_call(
        _tcC_body,
        grid=(N // RB,),
        in_specs=[
            pl.BlockSpec((NC, 1, RB, 1), lambda i: (0, i, 0, 0)),
            pl.BlockSpec((NC, RB, 128), lambda i: (0, i, 0)),
            pl.BlockSpec((RB, 128), lambda i: (i, 0)),
            pl.BlockSpec((1, 64), lambda i: (0, 0)),
            pl.BlockSpec((64, 64), lambda i: (0, 0)),
            pl.BlockSpec((1, 64), lambda i: (0, 0)),
        ],
        out_specs=pl.BlockSpec((RB, 64), lambda i: (i, 0)),
        out_shape=jax.ShapeDtypeStruct((N, 64), jnp.float32),
    )(cnt, g2, y2, b2, wl, bl)


def kernel(x, edge_index, W1, b1, W2, b2, Wl, bl):
    src3 = edge_index[0].reshape(NW, EPW)
    dst3 = edge_index[1].reshape(NW, EPW)
    cnt = _deg_counts(dst3)
    cnt = cnt[:, :N].reshape(NC, N // RB, RB, 1)
    y1t = _tcA(cnt, x, W1)
    g1 = _edge_sum(y1t, src3, dst3, 128)
    y2t = _tcB(cnt, g1, y1t, W2, Wl, b1.reshape(1, 128))
    g2 = _edge_sum(y2t, src3, dst3, 128)
    return _tcC(cnt, g2, y2t, b2.reshape(1, 64), Wl, bl.reshape(1, 64))


# speedup vs baseline: 26.4710x; 1.0433x over previous
import jax
import jax.numpy as jnp
from jax import lax
from jax.experimental import pallas as pl
from jax.experimental.pallas import tpu as pltpu
from jax.experimental.pallas import tpu_sc as plsc

N = 10000
E = 320000
NPAD = 10240
NC = 2
NS = 16
NW = NC * NS
EPW = E // NW
K = 40
NCHUNK = EPW // K
RPT = NPAD // NS
RB = 1000
NB = 5


def _mesh():
    return plsc.VectorSubcoreMesh(core_axis_name="c", subcore_axis_name="s")


def _deg_body(dst3_hbm, ones_hbm, zeros_hbm, out_hbm, ones_v, idx_d3, acc, sem_s):
    c = lax.axis_index("c")
    s = lax.axis_index("s")
    wid = s * NC + c
    rpt = NPAD // NS
    pltpu.sync_copy(zeros_hbm.at[pl.ds(s * rpt, rpt)], acc.at[pl.ds(s * rpt, rpt)])
    pltpu.sync_copy(ones_hbm, ones_v)
    pltpu.sync_copy(dst3_hbm.at[wid], idx_d3)
    plsc.subcore_barrier()

    def outer(g, carry):
        for b in range(NB):
            i = g * NB + b

            @pl.when(i >= NB)
            def _():
                pltpu.make_async_copy(
                    ones_v, acc.at[idx_d3.at[pl.ds((i - NB) * K, K)]], sem_s.at[b]).wait()

            pltpu.async_copy(ones_v, acc.at[idx_d3.at[pl.ds(i * K, K)]], sem_s.at[b], add=True)
        return carry

    lax.fori_loop(0, NCHUNK // NB, outer, 0)
    for b in range(NB):
        last = (NCHUNK // NB - 1) * NB + b
        pltpu.make_async_copy(
            ones_v, acc.at[idx_d3.at[pl.ds(last * K, K)]], sem_s.at[b]).wait()
    plsc.subcore_barrier()
    pltpu.sync_copy(acc.at[pl.ds(s * rpt, rpt)], out_hbm.at[c, pl.ds(s * rpt, rpt)])


def _deg_counts(dst3):
    ones = jnp.ones((K,), jnp.float32)
    zeros = jnp.zeros((NPAD,), jnp.float32)
    return pl.kernel(
        _deg_body,
        out_type=jax.ShapeDtypeStruct((NC, NPAD), jnp.float32),
        mesh=_mesh(),
        scratch_types=[
            pltpu.VMEM((K,), jnp.float32),
            pltpu.VMEM((EPW,), jnp.int32),
            pltpu.VMEM_SHARED((NPAD,), jnp.float32),
            pltpu.SemaphoreType.DMA((NB,)),
        ],
    )(dst3, ones, zeros)


def _prop_body(y_hbm, src3_hbm, dst3_hbm, out_hbm,
               idx_s3, idx_d3, rows, acc, sem_g, sem_s):
    c = lax.axis_index("c")
    s = lax.axis_index("s")
    wid = s * NC + c
    pltpu.async_copy(src3_hbm.at[wid], idx_s3, sem_g.at[0])
    pltpu.async_copy(dst3_hbm.at[wid], idx_d3, sem_g.at[1])
    z16 = jnp.zeros((16,), jnp.float32)

    def zrow(r, carry):
        for j in range(rows.shape[2] // 16):
            rows[0, r, pl.ds(j * 16, 16)] = z16
        return carry

    lax.fori_loop(0, K, zrow, 0)
    for k in range(RPT // K):
        pltpu.sync_copy(rows.at[0], acc.at[pl.ds(s * RPT + k * K, K)])
    pltpu.make_async_copy(src3_hbm.at[wid], idx_s3, sem_g.at[0]).wait()
    pltpu.make_async_copy(dst3_hbm.at[wid], idx_d3, sem_g.at[1]).wait()
    plsc.subcore_barrier()

    for b in range(NB):
        pltpu.async_copy(y_hbm.at[idx_s3.at[pl.ds(b * K, K)]], rows.at[b], sem_g.at[b])

    def outer(g, carry):
        for b in range(NB):
            i = g * NB + b
            jb = (b + NB - 1) % NB
            pltpu.make_async_copy(
                y_hbm.at[idx_s3.at[pl.ds(i * K, K)]], rows.at[b], sem_g.at[b]).wait()
            pltpu.async_copy(rows.at[b], acc.at[idx_d3.at[pl.ds(i * K, K)]], sem_s.at[b], add=True)
            j = i + NB - 1

            @pl.when((i >= 1) & (j <= NCHUNK - 1))
            def _():
                pltpu.make_async_copy(
                    rows.at[jb], acc.at[idx_d3.at[pl.ds((j - NB) * K, K)]], sem_s.at[jb]).wait()
                pltpu.async_copy(y_hbm.at[idx_s3.at[pl.ds(j * K, K)]], rows.at[jb], sem_g.at[jb])
        return carry

    lax.fori_loop(0, NCHUNK // NB, outer, 0)
    for b in range(NB):
        last = (NCHUNK // NB - 1) * NB + b
        pltpu.make_async_copy(
            rows.at[b], acc.at[idx_d3.at[pl.ds(last * K, K)]], sem_s.at[b]).wait()
    plsc.subcore_barrier()
    pltpu.sync_copy(acc.at[pl.ds(s * RPT, RPT), :], out_hbm.at[c, pl.ds(s * RPT, RPT), :])


def _edge_sum(y, src3, dst3, d):
    return pl.kernel(
        _prop_body,
        out_type=jax.ShapeDtypeStruct((NC, NPAD, d), jnp.float32),
        mesh=_mesh(),
        scratch_types=[
            pltpu.VMEM((EPW,), jnp.int32),
            pltpu.VMEM((EPW,), jnp.int32),
            pltpu.VMEM((NB, K, d), jnp.float32),
            pltpu.VMEM_SHARED((NPAD, d), jnp.float32),
            pltpu.SemaphoreType.DMA((NB,)),
            pltpu.SemaphoreType.DMA((NB,)),
        ],
    )(y, src3, dst3)


def _tcA_body(cnt_ref, x_ref, w1_ref, out_ref):
    dis = lax.rsqrt(cnt_ref[0, 0] + cnt_ref[1, 0] + 1.0)
    y = jnp.dot(x_ref[...], w1_ref[...], preferred_element_type=jnp.float32)
    out_ref[...] = y * dis


def _tcA(cnt, x, w1):
    return pl.pallas_call(
        _tcA_body,
        grid=(N // RB,),
        in_specs=[
            pl.BlockSpec((NC, 1, RB, 1), lambda i: (0, i, 0, 0)),
            pl.BlockSpec((RB, 128), lambda i: (i, 0)),
            pl.BlockSpec((128, 128), lambda i: (0, 0)),
        ],
        out_specs=pl.BlockSpec((RB, 128), lambda i: (i, 0)),
        out_shape=jax.ShapeDtypeStruct((N, 128), jnp.float32),
    )(cnt, x, w1)


def _tcB_body(cnt_ref, g_ref, y1_ref, w2_ref, wl_ref, b1_ref, out_ref):
    dis = lax.rsqrt(cnt_ref[0, 0] + cnt_ref[1, 0] + 1.0)
    p = (g_ref[0] + g_ref[1] + y1_ref[...]) * dis + b1_ref[...]
    h = jnp.maximum(p, 0.0)
    y2 = jnp.dot(jnp.dot(h, w2_ref[...], preferred_element_type=jnp.float32),
                 wl_ref[...], preferred_element_type=jnp.float32)
    out_ref[:, :64] = y2 * dis
    out_ref[:, 64:] = jnp.zeros((RB, 64), jnp.float32)


def _tcB(cnt, g1, y1, w2, wl, b1):
    return pl.pallas_call(
        _tcB_body,
        grid=(N // RB,),
        in_specs=[
            pl.BlockSpec((NC, 1, RB, 1), lambda i: (0, i, 0, 0)),
            pl.BlockSpec((NC, RB, 128), lambda i: (0, i, 0)),
            pl.BlockSpec((RB, 128), lambda i: (i, 0)),
            pl.BlockSpec((128, 64), lambda i: (0, 0)),
            pl.BlockSpec((64, 64), lambda i: (0, 0)),
            pl.BlockSpec((1, 128), lambda i: (0, 0)),
        ],
        out_specs=pl.BlockSpec((RB, 128), lambda i: (i, 0)),
        out_shape=jax.ShapeDtypeStruct((N, 128), jnp.float32),
    )(cnt, g1, y1, w2, wl, b1)


def _tcC_body(cnt_ref, g_ref, y2_ref, b2_ref, wl_ref, bl_ref, out_ref):
    dis = lax.rsqrt(cnt_ref[0, 0] + cnt_ref[1, 0] + 1.0)
    bias = jnp.dot(b2_ref[...], wl_ref[...], preferred_element_type=jnp.float32) + bl_ref[...]
    p = g_ref[0, :, :64] + g_ref[1, :, :64] + y2_ref[:, :64]
    out_ref[...] = p * dis + bias


def _tcC(cnt, g2, y2, b2, wl, bl):
    return pl.pallas_call(
        _tcC_body,
        grid=(N // RB,),
        in_specs=[
            pl.BlockSpec((NC, 1, RB, 1), lambda i: (0, i, 0, 0)),
            pl.BlockSpec((NC, RB, 128), lambda i: (0, i, 0)),
            pl.BlockSpec((RB, 128), lambda i: (i, 0)),
            pl.BlockSpec((1, 64), lambda i: (0, 0)),
            pl.BlockSpec((64, 64), lambda i: (0, 0)),
            pl.BlockSpec((1, 64), lambda i: (0, 0)),
        ],
        out_specs=pl.BlockSpec((RB, 64), lambda i: (i, 0)),
        out_shape=jax.ShapeDtypeStruct((N, 64), jnp.float32),
    )(cnt, g2, y2, b2, wl, bl)


def kernel(x, edge_index, W1, b1, W2, b2, Wl, bl):
    src3 = edge_index[0].reshape(NW, EPW)
    dst3 = edge_index[1].reshape(NW, EPW)
    cnt = _deg_counts(dst3)
    cnt = cnt[:, :N].reshape(NC, N // RB, RB, 1)
    y1t = _tcA(cnt, x, W1)
    g1 = _edge_sum(y1t, src3, dst3, 128)
    y2t = _tcB(cnt, g1, y1t, W2, Wl, b1.reshape(1, 128))
    g2 = _edge_sum(y2t, src3, dst3, 128)
    return _tcC(cnt, g2, y2t, b2.reshape(1, 64), Wl, bl.reshape(1, 64))
